# baseline (speedup 1.0000x reference)
import jax
import jax.numpy as jnp
from jax import lax
from jax.experimental import pallas as pl
from jax.experimental.pallas import tpu as pltpu
from jax.experimental.pallas import tpu_sc as plsc

E = 8
TILE = 128
NT = 24
TPAD = NT * TILE
SC_CORES = 2
SC_SUBCORES = 16
NW = SC_CORES * SC_SUBCORES


def _router_dispatch_body(x_ref, tt_ref, rw_ref, rb_ref,
                          pos_ref, te_ref, tv_ref, aux_ref):
    T, _ = x_ref.shape
    ttb = jnp.broadcast_to(tt_ref[...], (T, tt_ref.shape[1]))
    rin = jnp.concatenate([x_ref[...], ttb], axis=-1)
    logits = jnp.dot(rin, rw_ref[...], preferred_element_type=jnp.float32)
    logits = logits + rb_ref[...]
    m = jnp.max(logits, axis=1, keepdims=True)
    ex = jnp.exp(logits - m)
    probs = ex / jnp.sum(ex, axis=1, keepdims=True)
    lane = lax.broadcasted_iota(jnp.int32, (T, E), 1)
    eid = jnp.min(jnp.where(logits == m, lane, E), axis=1, keepdims=True)
    onehot = (lane == eid).astype(jnp.float32)
    counts = jnp.sum(onehot, axis=0, keepdims=True)
    imps = jnp.sum(probs, axis=0, keepdims=True)
    aux_ref[...] = (E / (T * T)) * jnp.sum(imps * counts, axis=(0, 1),
                                           keepdims=True)
    ci = counts.astype(jnp.int32)
    pc = ((ci + (TILE - 1)) // TILE) * TILE
    pcf = pc.astype(jnp.float32)
    er = lax.broadcasted_iota(jnp.int32, (E, E), 0)
    ec = lax.broadcasted_iota(jnp.int32, (E, E), 1)
    upper = (er < ec).astype(jnp.float32)
    off = jnp.dot(pcf, upper,
                  preferred_element_type=jnp.float32)
    ends = off + pcf
    CH = 128
    NC = T // CH
    r1 = lax.broadcasted_iota(jnp.int32, (CH, CH), 0)
    c1 = lax.broadcasted_iota(jnp.int32, (CH, CH), 1)
    sl1 = (c1 < r1).astype(jnp.float32)
    chunk_tots = []
    local_ranks = []
    for b in range(NC):
        ohb = onehot[b * CH:(b + 1) * CH, :]
        local_ranks.append(jnp.dot(sl1, ohb,
                                   preferred_element_type=jnp.float32))
        chunk_tots.append(jnp.sum(ohb, axis=0, keepdims=True))
    tots = jnp.concatenate(chunk_tots, axis=0)
    r2 = lax.broadcasted_iota(jnp.int32, (NC, NC), 0)
    c2 = lax.broadcasted_iota(jnp.int32, (NC, NC), 1)
    sl2 = (c2 < r2).astype(jnp.float32)
    carry = jnp.dot(sl2, tots,
                    preferred_element_type=jnp.float32)
    pos_rows = []
    for b in range(NC):
        slot = (local_ranks[b] + carry[b:b + 1, :] + off)
        ohb = onehot[b * CH:(b + 1) * CH, :]
        sel = jnp.transpose(slot * ohb)
        pos_rows.append(jnp.sum(sel, axis=0, keepdims=True))
    pos_ref[...] = jnp.concatenate(pos_rows, axis=0).astype(jnp.int32)
    tstart = (lax.broadcasted_iota(jnp.int32, (NT, E), 0) * TILE).astype(
        jnp.float32)
    ends_b = jnp.broadcast_to(ends, (NT, E))
    te = jnp.sum((ends_b <= tstart).astype(jnp.float32), axis=1,
                 keepdims=True).astype(jnp.int32)
    te_ref[...] = jnp.minimum(te, E - 1)
    total = jnp.sum(pcf, axis=1, keepdims=True)
    tv_ref[...] = (tstart[:, :1] < total).astype(jnp.int32)


def _ffn_body(te_ref, tv_ref, xs_ref, w1_ref, b1_ref, w2_ref, b2_ref,
              out_ref):
    i = pl.program_id(0)

    @pl.when(tv_ref[i] > 0)
    def _():
        e = te_ref[i]
        xt = xs_ref[...]
        h = jnp.dot(xt, w1_ref[0], preferred_element_type=jnp.float32)
        h = h + b1_ref[pl.ds(e, 1), :]
        h = 0.5 * h * (1.0 + lax.erf(h * (2.0 ** -0.5)))
        y = jnp.dot(h, w2_ref[0], preferred_element_type=jnp.float32)
        out_ref[...] = y + b2_ref[pl.ds(e, 1), :]


def _sc_scatter_body(x_hbm, pos_hbm, out_hbm, idx_v, rows_v, sem):
    chunk = idx_v.shape[0]
    wid = lax.axis_index("s") * SC_CORES + lax.axis_index("c")
    base = wid * chunk
    pltpu.sync_copy(pos_hbm.at[pl.ds(base, chunk)], idx_v)
    pltpu.sync_copy(x_hbm.at[pl.ds(base, chunk)], rows_v)
    pltpu.async_copy(rows_v, out_hbm.at[idx_v], sem).wait()


def _sc_gather_body(ys_hbm, pos_hbm, out_hbm, idx_v, rows_v, sem):
    chunk = idx_v.shape[0]
    wid = lax.axis_index("s") * SC_CORES + lax.axis_index("c")
    base = wid * chunk
    pltpu.sync_copy(pos_hbm.at[pl.ds(base, chunk)], idx_v)
    pltpu.async_copy(ys_hbm.at[idx_v], rows_v, sem).wait()
    pltpu.sync_copy(rows_v, out_hbm.at[pl.ds(base, chunk)])


def kernel(x, task_token, router_w, router_b, w1, b1, w2, b2):
    Bs, Ls, D = x.shape
    T = Bs * Ls
    F = w1.shape[-1]
    xt = x.reshape(T, D)
    rb = router_b.reshape(1, E)

    pos2, te2, tv2, aux = pl.pallas_call(
        _router_dispatch_body,
        out_shape=(
            jax.ShapeDtypeStruct((T // 128, 128), jnp.int32),
            jax.ShapeDtypeStruct((NT, 1), jnp.int32),
            jax.ShapeDtypeStruct((NT, 1), jnp.int32),
            jax.ShapeDtypeStruct((1, 1), jnp.float32),
        ),
    )(xt, task_token, router_w, rb)
    pos = pos2.reshape(T)
    te = te2.reshape(NT)
    tv = tv2.reshape(NT)

    chunk = T // NW
    mesh = plsc.VectorSubcoreMesh(core_axis_name="c", subcore_axis_name="s")
    xs = pl.kernel(
        _sc_scatter_body,
        out_type=jax.ShapeDtypeStruct((TPAD, D), jnp.float32),
        mesh=mesh,
        scratch_types=[
            pltpu.VMEM((chunk,), jnp.int32),
            pltpu.VMEM((chunk, D), jnp.float32),
            pltpu.SemaphoreType.DMA,
        ],
    )(xt, pos)

    grid_spec = pltpu.PrefetchScalarGridSpec(
        num_scalar_prefetch=2,
        grid=(NT,),
        in_specs=[
            pl.BlockSpec((TILE, D),
                         lambda i, te_r, tv_r: (i * tv_r[i], 0)),
            pl.BlockSpec((1, D, F),
                         lambda i, te_r, tv_r: (te_r[i], 0, 0)),
            pl.BlockSpec((E, F), lambda i, te_r, tv_r: (0, 0)),
            pl.BlockSpec((1, F, D),
                         lambda i, te_r, tv_r: (te_r[i], 0, 0)),
            pl.BlockSpec((E, D), lambda i, te_r, tv_r: (0, 0)),
        ],
        out_specs=pl.BlockSpec((TILE, D), lambda i, te_r, tv_r: (i, 0)),
    )
    ys = pl.pallas_call(
        _ffn_body,
        grid_spec=grid_spec,
        out_shape=jax.ShapeDtypeStruct((TPAD, D), jnp.float32),
    )(te, tv, xs, w1, b1, w2, b2)

    yt = pl.kernel(
        _sc_gather_body,
        out_type=jax.ShapeDtypeStruct((T, D), jnp.float32),
        mesh=mesh,
        scratch_types=[
            pltpu.VMEM((chunk,), jnp.int32),
            pltpu.VMEM((chunk, D), jnp.float32),
            pltpu.SemaphoreType.DMA,
        ],
    )(ys, pos)

    return yt.reshape(Bs, Ls, D), aux.reshape(())


# speedup vs baseline: 1.1013x; 1.1013x over previous
import jax
import jax.numpy as jnp
from jax import lax
from jax.experimental import pallas as pl
from jax.experimental.pallas import tpu as pltpu
from jax.experimental.pallas import tpu_sc as plsc

E = 8
TILE = 256
NT = 16
TPAD = NT * TILE
SC_CORES = 2
SC_SUBCORES = 16
NW = SC_CORES * SC_SUBCORES


def _router_dispatch_body(x_ref, tt_ref, rw_ref, rb_ref,
                          pos_ref, te_ref, tv_ref, aux_ref):
    T, _ = x_ref.shape
    ttb = jnp.broadcast_to(tt_ref[...], (T, tt_ref.shape[1]))
    rin = jnp.concatenate([x_ref[...], ttb], axis=-1)
    logits = jnp.dot(rin, rw_ref[...], preferred_element_type=jnp.float32)
    logits = logits + rb_ref[...]
    m = jnp.max(logits, axis=1, keepdims=True)
    ex = jnp.exp(logits - m)
    probs = ex / jnp.sum(ex, axis=1, keepdims=True)
    lane = lax.broadcasted_iota(jnp.int32, (T, E), 1)
    eid = jnp.min(jnp.where(logits == m, lane, E), axis=1, keepdims=True)
    onehot = (lane == eid).astype(jnp.float32)
    counts = jnp.sum(onehot, axis=0, keepdims=True)
    imps = jnp.sum(probs, axis=0, keepdims=True)
    aux_ref[...] = (E / (T * T)) * jnp.sum(imps * counts, axis=(0, 1),
                                           keepdims=True)
    ci = counts.astype(jnp.int32)
    pc = ((ci + (TILE - 1)) // TILE) * TILE
    pcf = pc.astype(jnp.float32)
    er = lax.broadcasted_iota(jnp.int32, (E, E), 0)
    ec = lax.broadcasted_iota(jnp.int32, (E, E), 1)
    upper = (er < ec).astype(jnp.float32)
    off = jnp.dot(pcf, upper,
                  preferred_element_type=jnp.float32)
    ends = off + pcf
    CH = 128
    NC = T // CH
    r1 = lax.broadcasted_iota(jnp.int32, (CH, CH), 0)
    c1 = lax.broadcasted_iota(jnp.int32, (CH, CH), 1)
    sl1 = (c1 < r1).astype(jnp.float32)
    chunk_tots = []
    local_ranks = []
    for b in range(NC):
        ohb = onehot[b * CH:(b + 1) * CH, :]
        local_ranks.append(jnp.dot(sl1, ohb,
                                   preferred_element_type=jnp.float32))
        chunk_tots.append(jnp.sum(ohb, axis=0, keepdims=True))
    tots = jnp.concatenate(chunk_tots, axis=0)
    r2 = lax.broadcasted_iota(jnp.int32, (NC, NC), 0)
    c2 = lax.broadcasted_iota(jnp.int32, (NC, NC), 1)
    sl2 = (c2 < r2).astype(jnp.float32)
    carry = jnp.dot(sl2, tots,
                    preferred_element_type=jnp.float32)
    pos_rows = []
    for b in range(NC):
        slot = (local_ranks[b] + carry[b:b + 1, :] + off)
        ohb = onehot[b * CH:(b + 1) * CH, :]
        sel = jnp.transpose(slot * ohb)
        pos_rows.append(jnp.sum(sel, axis=0, keepdims=True))
    pos_ref[...] = jnp.concatenate(pos_rows, axis=0).astype(jnp.int32)
    tstart = (lax.broadcasted_iota(jnp.int32, (NT, E), 0) * TILE).astype(
        jnp.float32)
    ends_b = jnp.broadcast_to(ends, (NT, E))
    te = jnp.sum((ends_b <= tstart).astype(jnp.float32), axis=1,
                 keepdims=True).astype(jnp.int32)
    te_ref[...] = jnp.minimum(te, E - 1)
    total = jnp.sum(pcf, axis=1, keepdims=True)
    tv_ref[...] = (tstart[:, :1] < total).astype(jnp.int32)


def _ffn_body(te_ref, tv_ref, xs_ref, w1_ref, b1_ref, w2_ref, b2_ref,
              out_ref):
    i = pl.program_id(0)

    @pl.when(tv_ref[i, 0] > 0)
    def _():
        e = te_ref[i, 0]
        xt = xs_ref[...]
        h = jnp.dot(xt, w1_ref[0], preferred_element_type=jnp.float32)
        h = h + b1_ref[pl.ds(e, 1), :]
        h = 0.5 * h * (1.0 + lax.erf(h * (2.0 ** -0.5)))
        y = jnp.dot(h, w2_ref[0], preferred_element_type=jnp.float32)
        out_ref[...] = y + b2_ref[pl.ds(e, 1), :]


def _sc_scatter_body(x_hbm, pos_hbm, out_hbm, idx_v, rows_v, sem):
    chunk = idx_v.shape[0]
    wid = lax.axis_index("s") * SC_CORES + lax.axis_index("c")
    base = wid * chunk
    pltpu.sync_copy(pos_hbm.at[pl.ds(base, chunk)], idx_v)
    pltpu.sync_copy(x_hbm.at[pl.ds(base, chunk)], rows_v)
    pltpu.async_copy(rows_v, out_hbm.at[idx_v], sem).wait()


def _sc_gather_body(ys_hbm, pos_hbm, out_hbm, idx_v, rows_v, sem):
    chunk = idx_v.shape[0]
    wid = lax.axis_index("s") * SC_CORES + lax.axis_index("c")
    base = wid * chunk
    pltpu.sync_copy(pos_hbm.at[pl.ds(base, chunk)], idx_v)
    pltpu.async_copy(ys_hbm.at[idx_v], rows_v, sem).wait()
    pltpu.sync_copy(rows_v, out_hbm.at[pl.ds(base, chunk)])


def kernel(x, task_token, router_w, router_b, w1, b1, w2, b2):
    Bs, Ls, D = x.shape
    T = Bs * Ls
    F = w1.shape[-1]
    xt = x.reshape(T, D)
    rb = router_b.reshape(1, E)

    pos2, te2, tv2, aux = pl.pallas_call(
        _router_dispatch_body,
        out_shape=(
            jax.ShapeDtypeStruct((T // 128, 128), jnp.int32),
            jax.ShapeDtypeStruct((NT, 1), jnp.int32),
            jax.ShapeDtypeStruct((NT, 1), jnp.int32),
            jax.ShapeDtypeStruct((1, 1), jnp.float32),
        ),
    )(xt, task_token, router_w, rb)
    pos = pos2.reshape(T)

    chunk = T // NW
    mesh = plsc.VectorSubcoreMesh(core_axis_name="c", subcore_axis_name="s")
    xs = pl.kernel(
        _sc_scatter_body,
        out_type=jax.ShapeDtypeStruct((TPAD, D), jnp.float32),
        mesh=mesh,
        scratch_types=[
            pltpu.VMEM((chunk,), jnp.int32),
            pltpu.VMEM((chunk, D), jnp.float32),
            pltpu.SemaphoreType.DMA,
        ],
    )(xt, pos)

    grid_spec = pltpu.PrefetchScalarGridSpec(
        num_scalar_prefetch=2,
        grid=(NT,),
        in_specs=[
            pl.BlockSpec((TILE, D),
                         lambda i, te_r, tv_r: (i * tv_r[i, 0], 0)),
            pl.BlockSpec((1, D, F),
                         lambda i, te_r, tv_r: (te_r[i, 0], 0, 0)),
            pl.BlockSpec((E, F), lambda i, te_r, tv_r: (0, 0)),
            pl.BlockSpec((1, F, D),
                         lambda i, te_r, tv_r: (te_r[i, 0], 0, 0)),
            pl.BlockSpec((E, D), lambda i, te_r, tv_r: (0, 0)),
        ],
        out_specs=pl.BlockSpec((TILE, D), lambda i, te_r, tv_r: (i, 0)),
    )
    ys = pl.pallas_call(
        _ffn_body,
        grid_spec=grid_spec,
        out_shape=jax.ShapeDtypeStruct((TPAD, D), jnp.float32),
    )(te2, tv2, xs, w1, b1, w2, b2)

    yt = pl.kernel(
        _sc_gather_body,
        out_type=jax.ShapeDtypeStruct((T, D), jnp.float32),
        mesh=mesh,
        scratch_types=[
            pltpu.VMEM((chunk,), jnp.int32),
            pltpu.VMEM((chunk, D), jnp.float32),
            pltpu.SemaphoreType.DMA,
        ],
    )(ys, pos)

    return yt.reshape(Bs, Ls, D), aux.reshape(())


# speedup vs baseline: 1.1050x; 1.0033x over previous
import jax
import jax.numpy as jnp
from jax import lax
from jax.experimental import pallas as pl
from jax.experimental.pallas import tpu as pltpu
from jax.experimental.pallas import tpu_sc as plsc

E = 8
TILE = 256
NT = 16
TPAD = NT * TILE
SC_CORES = 2
SC_SUBCORES = 16
NW = SC_CORES * SC_SUBCORES


def _router_dispatch_body(x_ref, tt_ref, rw_ref, rb_ref,
                          pos_ref, te_ref, tv_ref, aux_ref):
    T, _ = x_ref.shape
    ttb = jnp.broadcast_to(tt_ref[...], (T, tt_ref.shape[1]))
    rin = jnp.concatenate([x_ref[...], ttb], axis=-1)
    logits = jnp.dot(rin, rw_ref[...], preferred_element_type=jnp.float32)
    logits = logits + rb_ref[...]
    m = jnp.max(logits, axis=1, keepdims=True)
    ex = jnp.exp(logits - m)
    probs = ex / jnp.sum(ex, axis=1, keepdims=True)
    lane = lax.broadcasted_iota(jnp.int32, (T, E), 1)
    eid = jnp.min(jnp.where(logits == m, lane, E), axis=1, keepdims=True)
    onehot = (lane == eid).astype(jnp.float32)
    counts = jnp.sum(onehot, axis=0, keepdims=True)
    imps = jnp.sum(probs, axis=0, keepdims=True)
    aux_ref[...] = (E / (T * T)) * jnp.sum(imps * counts, axis=(0, 1),
                                           keepdims=True)
    ci = counts.astype(jnp.int32)
    pc = ((ci + (TILE - 1)) // TILE) * TILE
    pcf = pc.astype(jnp.float32)
    er = lax.broadcasted_iota(jnp.int32, (E, E), 0)
    ec = lax.broadcasted_iota(jnp.int32, (E, E), 1)
    upper = (er < ec).astype(jnp.float32)
    off = jnp.dot(pcf, upper,
                  preferred_element_type=jnp.float32)
    ends = off + pcf
    CH = 128
    NC = T // CH
    r1 = lax.broadcasted_iota(jnp.int32, (CH, CH), 0)
    c1 = lax.broadcasted_iota(jnp.int32, (CH, CH), 1)
    sl1 = (c1 < r1).astype(jnp.float32)
    chunk_tots = []
    local_ranks = []
    for b in range(NC):
        ohb = onehot[b * CH:(b + 1) * CH, :]
        local_ranks.append(jnp.dot(sl1, ohb,
                                   preferred_element_type=jnp.float32))
        chunk_tots.append(jnp.sum(ohb, axis=0, keepdims=True))
    tots = jnp.concatenate(chunk_tots, axis=0)
    r2 = lax.broadcasted_iota(jnp.int32, (NC, NC), 0)
    c2 = lax.broadcasted_iota(jnp.int32, (NC, NC), 1)
    sl2 = (c2 < r2).astype(jnp.float32)
    carry = jnp.dot(sl2, tots,
                    preferred_element_type=jnp.float32)
    pos_rows = []
    for b in range(NC):
        slot = (local_ranks[b] + carry[b:b + 1, :] + off)
        ohb = onehot[b * CH:(b + 1) * CH, :]
        sel = jnp.transpose(slot * ohb)
        pos_rows.append(jnp.sum(sel, axis=0, keepdims=True))
    pos_ref[...] = jnp.concatenate(pos_rows, axis=0).astype(jnp.int32)
    tstart = (lax.broadcasted_iota(jnp.int32, (NT, E), 0) * TILE).astype(
        jnp.float32)
    ends_b = jnp.broadcast_to(ends, (NT, E))
    te = jnp.sum((ends_b <= tstart).astype(jnp.float32), axis=1,
                 keepdims=True).astype(jnp.int32)
    te_ref[...] = jnp.minimum(te, E - 1)
    total = jnp.sum(pcf, axis=1, keepdims=True)
    tv_ref[...] = (tstart[:, :1] < total).astype(jnp.int32)


def _ffn_body(te_ref, tv_ref, xs_ref, w1_ref, b1_ref, w2_ref, b2_ref,
              out_ref):
    i = pl.program_id(0)

    @pl.when(tv_ref[i, 0] > 0)
    def _():
        e = te_ref[i, 0]
        xt = xs_ref[...]
        h = jnp.dot(xt, w1_ref[0], preferred_element_type=jnp.float32)
        h = h + b1_ref[pl.ds(e, 1), :]
        h = 0.5 * h * (1.0 + lax.erf(h * (2.0 ** -0.5)))
        y = jnp.dot(h, w2_ref[0], preferred_element_type=jnp.float32)
        out_ref[...] = y + b2_ref[pl.ds(e, 1), :]


def _sc_scatter_body(x_hbm, pos_hbm, out_hbm, idx_v, rows_v, sem0, sem1):
    chunk = idx_v.shape[0]
    half = chunk // 2
    wid = lax.axis_index("s") * SC_CORES + lax.axis_index("c")
    base = wid * chunk
    pltpu.sync_copy(pos_hbm.at[pl.ds(base, chunk)], idx_v)
    cp0 = pltpu.async_copy(x_hbm.at[pl.ds(base, half)],
                           rows_v.at[pl.ds(0, half)], sem0)
    cp1 = pltpu.async_copy(x_hbm.at[pl.ds(base + half, half)],
                           rows_v.at[pl.ds(half, half)], sem1)
    cp0.wait()
    sc0 = pltpu.async_copy(rows_v.at[pl.ds(0, half)],
                           out_hbm.at[idx_v.at[pl.ds(0, half)]], sem0)
    cp1.wait()
    sc1 = pltpu.async_copy(rows_v.at[pl.ds(half, half)],
                           out_hbm.at[idx_v.at[pl.ds(half, half)]], sem1)
    sc0.wait()
    sc1.wait()


def _sc_gather_body(ys_hbm, pos_hbm, out_hbm, idx_v, rows_v, sem0, sem1):
    chunk = idx_v.shape[0]
    half = chunk // 2
    wid = lax.axis_index("s") * SC_CORES + lax.axis_index("c")
    base = wid * chunk
    pltpu.sync_copy(pos_hbm.at[pl.ds(base, chunk)], idx_v)
    g0 = pltpu.async_copy(ys_hbm.at[idx_v.at[pl.ds(0, half)]],
                          rows_v.at[pl.ds(0, half)], sem0)
    g1 = pltpu.async_copy(ys_hbm.at[idx_v.at[pl.ds(half, half)]],
                          rows_v.at[pl.ds(half, half)], sem1)
    g0.wait()
    cp0 = pltpu.async_copy(rows_v.at[pl.ds(0, half)],
                           out_hbm.at[pl.ds(base, half)], sem0)
    g1.wait()
    cp1 = pltpu.async_copy(rows_v.at[pl.ds(half, half)],
                           out_hbm.at[pl.ds(base + half, half)], sem1)
    cp0.wait()
    cp1.wait()


def kernel(x, task_token, router_w, router_b, w1, b1, w2, b2):
    Bs, Ls, D = x.shape
    T = Bs * Ls
    F = w1.shape[-1]
    xt = x.reshape(T, D)
    rb = router_b.reshape(1, E)

    pos2, te2, tv2, aux = pl.pallas_call(
        _router_dispatch_body,
        out_shape=(
            jax.ShapeDtypeStruct((T // 128, 128), jnp.int32),
            jax.ShapeDtypeStruct((NT, 1), jnp.int32),
            jax.ShapeDtypeStruct((NT, 1), jnp.int32),
            jax.ShapeDtypeStruct((1, 1), jnp.float32),
        ),
    )(xt, task_token, router_w, rb)
    pos = pos2.reshape(T)

    chunk = T // NW
    mesh = plsc.VectorSubcoreMesh(core_axis_name="c", subcore_axis_name="s")
    xs = pl.kernel(
        _sc_scatter_body,
        out_type=jax.ShapeDtypeStruct((TPAD, D), jnp.float32),
        mesh=mesh,
        scratch_types=[
            pltpu.VMEM((chunk,), jnp.int32),
            pltpu.VMEM((chunk, D), jnp.float32),
            pltpu.SemaphoreType.DMA,
            pltpu.SemaphoreType.DMA,
        ],
    )(xt, pos)

    grid_spec = pltpu.PrefetchScalarGridSpec(
        num_scalar_prefetch=2,
        grid=(NT,),
        in_specs=[
            pl.BlockSpec((TILE, D),
                         lambda i, te_r, tv_r: (i * tv_r[i, 0], 0)),
            pl.BlockSpec((1, D, F),
                         lambda i, te_r, tv_r: (te_r[i, 0], 0, 0)),
            pl.BlockSpec((E, F), lambda i, te_r, tv_r: (0, 0)),
            pl.BlockSpec((1, F, D),
                         lambda i, te_r, tv_r: (te_r[i, 0], 0, 0)),
            pl.BlockSpec((E, D), lambda i, te_r, tv_r: (0, 0)),
        ],
        out_specs=pl.BlockSpec((TILE, D), lambda i, te_r, tv_r: (i, 0)),
    )
    ys = pl.pallas_call(
        _ffn_body,
        grid_spec=grid_spec,
        out_shape=jax.ShapeDtypeStruct((TPAD, D), jnp.float32),
    )(te2, tv2, xs, w1, b1, w2, b2)

    yt = pl.kernel(
        _sc_gather_body,
        out_type=jax.ShapeDtypeStruct((T, D), jnp.float32),
        mesh=mesh,
        scratch_types=[
            pltpu.VMEM((chunk,), jnp.int32),
            pltpu.VMEM((chunk, D), jnp.float32),
            pltpu.SemaphoreType.DMA,
            pltpu.SemaphoreType.DMA,
        ],
    )(ys, pos)

    return yt.reshape(Bs, Ls, D), aux.reshape(())
